# baseline (speedup 1.0000x reference)
import functools

import jax
import jax.numpy as jnp
from jax import lax
from jax.experimental import pallas as pl
from jax.experimental.pallas import tpu as pltpu
from jax.experimental.pallas import tpu_sc as plsc

B = 8
T = 512
D = 384
F = 1600
TBL = B * T
PAD_ROWS = 8
ZERO_ROW = TBL

NC, NS = 2, 16
NW = NC * NS
ROWS_PER_W = (B * F) // NW
CHUNK = 80
NCH = ROWS_PER_W // CHUNK


def _idx_body(pd_ref, idx_ref):
    b = pl.program_id(0)
    d = jnp.round(jnp.clip(pd_ref[...], 1.0, 20.0)).reshape(1, T)
    rows = lax.broadcasted_iota(jnp.int32, (T, T), 0)
    cols = lax.broadcasted_iota(jnp.int32, (T, T), 1)
    tri = (rows <= cols).astype(jnp.float32)
    cum = jnp.dot(d, tri, preferred_element_type=jnp.float32)
    pos = lax.broadcasted_iota(jnp.int32, (F, T), 0).astype(jnp.float32)
    m = (pos >= cum).astype(jnp.float32)
    tok = jnp.dot(m, jnp.ones((T, 1), jnp.float32),
                  preferred_element_type=jnp.float32)
    raw = tok.astype(jnp.int32)
    flat = jnp.where(raw < T, b * T + raw, ZERO_ROW)
    idx_ref[...] = flat.reshape(1, F, 1)


_idx_call = pl.pallas_call(
    _idx_body,
    grid=(B,),
    in_specs=[pl.BlockSpec((1, 1, T), lambda b: (b, 0, 0))],
    out_specs=pl.BlockSpec((1, F, 1), lambda b: (b, 0, 0)),
    out_shape=jax.ShapeDtypeStruct((B, F, 1), jnp.int32),
)


_sc_mesh = plsc.VectorSubcoreMesh(core_axis_name="c", subcore_axis_name="s")


@functools.partial(
    pl.kernel,
    mesh=_sc_mesh,
    out_type=jax.ShapeDtypeStruct((B * F, D), jnp.float32),
    scratch_types=[
        pltpu.VMEM((ROWS_PER_W,), jnp.int32),
        pltpu.VMEM((CHUNK, D), jnp.float32),
        pltpu.VMEM((CHUNK, D), jnp.float32),
        pltpu.SemaphoreType.DMA,
        pltpu.SemaphoreType.DMA,
        pltpu.SemaphoreType.DMA,
        pltpu.SemaphoreType.DMA,
    ],
)
def _sc_gather(table_hbm, idx_hbm, out_hbm, idx_v, buf0, buf1,
               gsem0, gsem1, ssem0, ssem1):
    wid = lax.axis_index("s") * NC + lax.axis_index("c")
    base = wid * ROWS_PER_W
    pltpu.sync_copy(idx_hbm.at[pl.ds(base, ROWS_PER_W)], idx_v)
    bufs = (buf0, buf1)
    gsems = (gsem0, gsem1)
    ssems = (ssem0, ssem1)
    gcp = [None, None]
    scp = [None, None]
    gcp[0] = pltpu.async_copy(
        table_hbm.at[idx_v.at[pl.ds(0, CHUNK)]], buf0, gsem0)
    for c in range(NCH):
        nxt = c + 1
        if nxt < NCH:
            if c >= 1:
                scp[nxt % 2].wait()
            gcp[nxt % 2] = pltpu.async_copy(
                table_hbm.at[idx_v.at[pl.ds(nxt * CHUNK, CHUNK)]],
                bufs[nxt % 2], gsems[nxt % 2])
        gcp[c % 2].wait()
        scp[c % 2] = pltpu.async_copy(
            bufs[c % 2], out_hbm.at[pl.ds(base + c * CHUNK, CHUNK)],
            ssems[c % 2])
    scp[(NCH - 2) % 2].wait()
    scp[(NCH - 1) % 2].wait()


def kernel(batch, predicted_durations):
    pd = predicted_durations.reshape(B, 1, T)
    idx = _idx_call(pd)
    table = jnp.concatenate(
        [batch.reshape(TBL, D), jnp.zeros((PAD_ROWS, D), jnp.float32)], axis=0)
    out = _sc_gather(table, idx.reshape(B * F))
    return out.reshape(B, F, D)


# speedup vs baseline: 1.0241x; 1.0241x over previous
import functools

import jax
import jax.numpy as jnp
from jax import lax
from jax.experimental import pallas as pl
from jax.experimental.pallas import tpu as pltpu
from jax.experimental.pallas import tpu_sc as plsc

B = 8
T = 512
D = 384
F = 1600
TBL = B * T
ZERO_ROW = TBL

NC, NS = 2, 16
NW = NC * NS
ROWS_PER_W = (B * F) // NW
CHUNK = 80
NCH = ROWS_PER_W // CHUNK


def _idx_body(pd_ref, batch_ref, idx_ref, table_ref):
    b = pl.program_id(0)

    @pl.when(b < B)
    def _():
        d = jnp.round(jnp.clip(pd_ref[...], 1.0, 20.0)).reshape(1, T)
        rows = lax.broadcasted_iota(jnp.int32, (T, T), 0)
        cols = lax.broadcasted_iota(jnp.int32, (T, T), 1)
        tri = (rows <= cols).astype(jnp.float32)
        cum = jnp.dot(d, tri, preferred_element_type=jnp.float32)
        pos = lax.broadcasted_iota(jnp.int32, (F, T), 0).astype(jnp.float32)
        m = (pos >= cum).astype(jnp.float32)
        tok = jnp.dot(m, jnp.ones((T, 1), jnp.float32),
                      preferred_element_type=jnp.float32)
        raw = tok.astype(jnp.int32)
        flat = jnp.where(raw < T, b * T + raw, ZERO_ROW)
        idx_ref[...] = flat.reshape(1, F, 1)
        table_ref[...] = batch_ref[...]

    @pl.when(b == B)
    def _():
        table_ref[...] = jnp.zeros_like(table_ref)


def _clampb(b):
    return jnp.minimum(b, B - 1)


_idx_call = pl.pallas_call(
    _idx_body,
    grid=(B + 1,),
    in_specs=[
        pl.BlockSpec((1, 1, T), lambda b: (_clampb(b), 0, 0)),
        pl.BlockSpec((1, T, D), lambda b: (_clampb(b), 0, 0)),
    ],
    out_specs=[
        pl.BlockSpec((1, F, 1), lambda b: (_clampb(b), 0, 0)),
        pl.BlockSpec((1, T, D), lambda b: (b, 0, 0)),
    ],
    out_shape=[
        jax.ShapeDtypeStruct((B, F, 1), jnp.int32),
        jax.ShapeDtypeStruct((B + 1, T, D), jnp.float32),
    ],
)


_sc_mesh = plsc.VectorSubcoreMesh(core_axis_name="c", subcore_axis_name="s")


@functools.partial(
    pl.kernel,
    mesh=_sc_mesh,
    out_type=jax.ShapeDtypeStruct((B * F, D), jnp.float32),
    scratch_types=[
        pltpu.VMEM((ROWS_PER_W,), jnp.int32),
        pltpu.VMEM((CHUNK, D), jnp.float32),
        pltpu.VMEM((CHUNK, D), jnp.float32),
        pltpu.SemaphoreType.DMA,
        pltpu.SemaphoreType.DMA,
        pltpu.SemaphoreType.DMA,
        pltpu.SemaphoreType.DMA,
    ],
)
def _sc_gather(table_hbm, idx_hbm, out_hbm, idx_v, buf0, buf1,
               gsem0, gsem1, ssem0, ssem1):
    wid = lax.axis_index("s") * NC + lax.axis_index("c")
    base = wid * ROWS_PER_W
    pltpu.sync_copy(idx_hbm.at[pl.ds(base, ROWS_PER_W)], idx_v)
    bufs = (buf0, buf1)
    gsems = (gsem0, gsem1)
    ssems = (ssem0, ssem1)
    gcp = [None, None]
    scp = [None, None]
    gcp[0] = pltpu.async_copy(
        table_hbm.at[idx_v.at[pl.ds(0, CHUNK)]], buf0, gsem0)
    for c in range(NCH):
        nxt = c + 1
        if nxt < NCH:
            if c >= 1:
                scp[nxt % 2].wait()
            gcp[nxt % 2] = pltpu.async_copy(
                table_hbm.at[idx_v.at[pl.ds(nxt * CHUNK, CHUNK)]],
                bufs[nxt % 2], gsems[nxt % 2])
        gcp[c % 2].wait()
        scp[c % 2] = pltpu.async_copy(
            bufs[c % 2], out_hbm.at[pl.ds(base + c * CHUNK, CHUNK)],
            ssems[c % 2])
    scp[(NCH - 2) % 2].wait()
    scp[(NCH - 1) % 2].wait()


def kernel(batch, predicted_durations):
    pd = predicted_durations.reshape(B, 1, T)
    idx, table = _idx_call(pd, batch)
    out = _sc_gather(table.reshape((B + 1) * T, D), idx.reshape(B * F))
    return out.reshape(B, F, D)


# speedup vs baseline: 1.0413x; 1.0168x over previous
import functools

import jax
import jax.numpy as jnp
from jax import lax
from jax.experimental import pallas as pl
from jax.experimental.pallas import tpu as pltpu
from jax.experimental.pallas import tpu_sc as plsc

B = 8
T = 512
D = 384
F = 1600
TBL = B * T
ZERO_ROW = TBL

NC, NS = 2, 16
NW = NC * NS
ROWS_PER_W = (B * F) // NW
CHUNK = 80
NCH = ROWS_PER_W // CHUNK


def _idx_body(pd_ref, batch_ref, idx_ref, table_ref):
    b = pl.program_id(0)

    @pl.when(b < B)
    def _():
        d = jnp.round(jnp.clip(pd_ref[...], 1.0, 20.0)).reshape(T, 1)
        rows = lax.broadcasted_iota(jnp.int32, (T, T), 0)
        cols = lax.broadcasted_iota(jnp.int32, (T, T), 1)
        tril = (rows >= cols).astype(jnp.float32)
        cum = jnp.dot(tril, d, preferred_element_type=jnp.float32)
        pos = lax.broadcasted_iota(jnp.int32, (T, F), 1).astype(jnp.float32)
        m = (pos >= cum).astype(jnp.float32)
        tok = jnp.dot(jnp.ones((1, T), jnp.float32), m,
                      preferred_element_type=jnp.float32)
        raw = tok.astype(jnp.int32)
        flat = jnp.where(raw < T, b * T + raw, ZERO_ROW)
        idx_ref[...] = flat.reshape(1, 1, F)
        table_ref[...] = batch_ref[...]

    @pl.when(b == B)
    def _():
        table_ref[...] = jnp.zeros_like(table_ref)


def _clampb(b):
    return jnp.minimum(b, B - 1)


_idx_call = pl.pallas_call(
    _idx_body,
    grid=(B + 1,),
    in_specs=[
        pl.BlockSpec((1, T, 1), lambda b: (_clampb(b), 0, 0)),
        pl.BlockSpec((1, T, D), lambda b: (_clampb(b), 0, 0)),
    ],
    out_specs=[
        pl.BlockSpec((1, 1, F), lambda b: (_clampb(b), 0, 0)),
        pl.BlockSpec((1, T, D), lambda b: (b, 0, 0)),
    ],
    out_shape=[
        jax.ShapeDtypeStruct((B, 1, F), jnp.int32),
        jax.ShapeDtypeStruct((B + 1, T, D), jnp.float32),
    ],
)


_sc_mesh = plsc.VectorSubcoreMesh(core_axis_name="c", subcore_axis_name="s")


@functools.partial(
    pl.kernel,
    mesh=_sc_mesh,
    out_type=jax.ShapeDtypeStruct((B * F, D), jnp.float32),
    scratch_types=[
        pltpu.VMEM((ROWS_PER_W,), jnp.int32),
        pltpu.VMEM((CHUNK, D), jnp.float32),
        pltpu.VMEM((CHUNK, D), jnp.float32),
        pltpu.SemaphoreType.DMA,
        pltpu.SemaphoreType.DMA,
        pltpu.SemaphoreType.DMA,
        pltpu.SemaphoreType.DMA,
    ],
)
def _sc_gather(table_hbm, idx_hbm, out_hbm, idx_v, buf0, buf1,
               gsem0, gsem1, ssem0, ssem1):
    wid = lax.axis_index("s") * NC + lax.axis_index("c")
    base = wid * ROWS_PER_W
    pltpu.sync_copy(idx_hbm.at[pl.ds(base, ROWS_PER_W)], idx_v)
    bufs = (buf0, buf1)
    gsems = (gsem0, gsem1)
    ssems = (ssem0, ssem1)
    gcp = [None, None]
    scp = [None, None]
    gcp[0] = pltpu.async_copy(
        table_hbm.at[idx_v.at[pl.ds(0, CHUNK)]], buf0, gsem0)
    for c in range(NCH):
        nxt = c + 1
        if nxt < NCH:
            if c >= 1:
                scp[nxt % 2].wait()
            gcp[nxt % 2] = pltpu.async_copy(
                table_hbm.at[idx_v.at[pl.ds(nxt * CHUNK, CHUNK)]],
                bufs[nxt % 2], gsems[nxt % 2])
        gcp[c % 2].wait()
        scp[c % 2] = pltpu.async_copy(
            bufs[c % 2], out_hbm.at[pl.ds(base + c * CHUNK, CHUNK)],
            ssems[c % 2])
    scp[(NCH - 2) % 2].wait()
    scp[(NCH - 1) % 2].wait()


def kernel(batch, predicted_durations):
    idx, table = _idx_call(predicted_durations, batch)
    out = _sc_gather(table.reshape((B + 1) * T, D), idx.reshape(B * F))
    return out.reshape(B, F, D)
